# baseline (speedup 1.0000x reference)
import functools

import jax
import jax.numpy as jnp
from jax import lax
from jax.experimental import pallas as pl
from jax.experimental.pallas import tpu as pltpu
from jax.experimental.pallas import tpu_sc as plsc

_B, _T, _C = 128, 196, 768
_M = 4
_S = _T // _M
_ROW = _S * _C
_L = _B * _M


def _sc_permute(x):
    info = plsc.get_sparse_core_info()
    nw = info.num_cores * info.num_subcores
    rows_per_w = _L // nw
    mesh = plsc.VectorSubcoreMesh(core_axis_name="c", subcore_axis_name="s")

    @functools.partial(
        pl.kernel,
        out_type=jax.ShapeDtypeStruct((_L, _ROW), jnp.float32),
        mesh=mesh,
        scratch_types=[pltpu.SemaphoreType.DMA],
    )
    def k(x_hbm, out_hbm, sem):
        wid = lax.axis_index("s") * info.num_cores + lax.axis_index("c")
        base = wid * rows_per_w
        copies = []
        for j in range(rows_per_w):
            i = base + j
            src = lax.rem(i + lax.rem(i, _M) * _M, _L)
            c = pltpu.make_async_copy(x_hbm.at[src], out_hbm.at[i], sem)
            c.start()
            copies.append(c)
        for c in copies:
            c.wait()

    return k(x)


def kernel(patches, m):
    del m
    rows = patches.reshape(_L, _ROW)
    mixed = _sc_permute(rows).reshape(_B, _T, _C)
    ids_b = jnp.arange(_B).reshape(-1, 1)
    target = (ids_b + jnp.arange(_M)) % _B
    mix_target = (ids_b - _M + 1 + jnp.arange(_M * 2 - 1) + _B) % _B
    return (mixed, target, mix_target)


# speedup vs baseline: 7.5587x; 7.5587x over previous
import functools

import jax
import jax.numpy as jnp
from jax import lax
from jax.experimental import pallas as pl
from jax.experimental.pallas import tpu as pltpu
from jax.experimental.pallas import tpu_sc as plsc

_B, _T, _C = 128, 196, 768
_M = 4
_S = _T // _M
_ROW = _S * _C
_L = _B * _M


def _sc_permute(x):
    info = plsc.get_sparse_core_info()
    nw = info.num_cores * info.num_subcores
    rows_per_w = _L // nw
    mesh = plsc.VectorSubcoreMesh(core_axis_name="c", subcore_axis_name="s")

    @functools.partial(
        pl.kernel,
        out_type=jax.ShapeDtypeStruct((_L, _ROW), jnp.float32),
        mesh=mesh,
        scratch_types=[
            pltpu.VMEM((2, _ROW), jnp.float32),
            pltpu.SemaphoreType.DMA,
            pltpu.SemaphoreType.DMA,
        ],
    )
    def k(x_hbm, out_hbm, buf, sem_in, sem_out):
        wid = lax.axis_index("s") * info.num_cores + lax.axis_index("c")
        base = wid * rows_per_w

        def in_copy(j):
            i = base + j
            src = lax.rem(i + lax.rem(i, _M) * _M, _L)
            return pltpu.make_async_copy(x_hbm.at[src], buf.at[j % 2], sem_in)

        def out_copy(j):
            return pltpu.make_async_copy(buf.at[j % 2], out_hbm.at[base + j],
                                         sem_out)

        ins = [in_copy(j) for j in range(rows_per_w)]
        outs = [out_copy(j) for j in range(rows_per_w)]
        ins[0].start()
        for j in range(rows_per_w):
            ins[j].wait()
            if j >= 1:
                outs[j - 1].wait()
            if j + 1 < rows_per_w:
                ins[j + 1].start()
            outs[j].start()
        outs[rows_per_w - 1].wait()

    return k(x)


def kernel(patches, m):
    del m
    rows = patches.reshape(_L, _ROW)
    mixed = _sc_permute(rows).reshape(_B, _T, _C)
    ids_b = jnp.arange(_B).reshape(-1, 1)
    target = (ids_b + jnp.arange(_M)) % _B
    mix_target = (ids_b - _M + 1 + jnp.arange(_M * 2 - 1) + _B) % _B
    return (mixed, target, mix_target)


# speedup vs baseline: 13.4026x; 1.7732x over previous
import functools

import jax
import jax.numpy as jnp
from jax import lax
from jax.experimental import pallas as pl
from jax.experimental.pallas import tpu as pltpu
from jax.experimental.pallas import tpu_sc as plsc

_B, _T, _C = 128, 196, 768
_M = 4
_S = _T // _M
_NV = _C // 16
_TAIL = _T % 8
_TA = _T - _TAIL

_ROFF = (0, 48, 96, 144)
_RLEN = (56, 56, 56, 48)
_WOFF = (0, 48, 96, 144)
_WLEN = 48
_NMOVE = (0, 1, 2, 3)
_RMAX = 56


def _sc_permute(x):
    info = plsc.get_sparse_core_info()
    nw = info.num_cores * info.num_subcores
    g_per_w = _B // nw
    mesh = plsc.VectorSubcoreMesh(core_axis_name="c", subcore_axis_name="s")

    @functools.partial(
        pl.kernel,
        out_type=jax.ShapeDtypeStruct((_B, _T, _C), jnp.float32),
        mesh=mesh,
        scratch_types=[
            pltpu.VMEM((3, _RMAX, _C), jnp.float32),
            pltpu.SemaphoreType.DMA,
            pltpu.SemaphoreType.DMA,
        ],
    )
    def k(x_hbm, out_hbm, ring, sem_in, sem_out):
        wid = lax.axis_index("s") * info.num_cores + lax.axis_index("c")
        g0 = wid * g_per_w
        chunks = [(dg, mm) for dg in range(g_per_w) for mm in range(_M)]
        n = len(chunks)

        def in_copy(j):
            dg, mm = chunks[j]
            sg = lax.rem(g0 + dg + mm, _B)
            return pltpu.make_async_copy(
                x_hbm.at[sg, pl.ds(_ROFF[mm], _RLEN[mm])],
                ring.at[j % 3, pl.ds(0, _RLEN[mm])], sem_in)

        def out_copy(j):
            dg, mm = chunks[j]
            return pltpu.make_async_copy(
                ring.at[j % 3, pl.ds(0, _WLEN)],
                out_hbm.at[g0 + dg, pl.ds(_WOFF[mm], _WLEN)], sem_out)

        def head_moves(j):
            mm = chunks[j][1]
            for t in range(_NMOVE[mm]):
                for v in range(_NV):
                    ring[j % 3, t, pl.ds(v * 16, 16)] = (
                        ring[(j - 1) % 3, 48 + t, pl.ds(v * 16, 16)])

        ins = [in_copy(j) for j in range(n)]
        outs = [out_copy(j) for j in range(n)]
        ins[0].start()
        ins[1].start()
        for j in range(n):
            ins[j].wait()
            head_moves(j)
            outs[j].start()
            if j >= 1:
                outs[j - 1].wait()
            if j + 2 < n:
                ins[j + 2].start()
        outs[n - 1].wait()

    return k(x)


def _tc_tail_body(main_ref, src_ref, out_ref):
    out_ref[0:_B - 3] = src_ref[3:_B]
    out_ref[_B - 3:_B] = src_ref[0:3]


def _tc_tail(main, patches):
    return pl.pallas_call(
        _tc_tail_body,
        grid=(1,),
        in_specs=[
            pl.BlockSpec((1, 8, 128), lambda i: (0, 0, 0)),
            pl.BlockSpec((_B, 8, _C), lambda i: (0, _TA // 8, 0)),
        ],
        out_specs=pl.BlockSpec((_B, 8, _C), lambda i: (0, _TA // 8, 0)),
        out_shape=jax.ShapeDtypeStruct((_B, _T, _C), jnp.float32),
        input_output_aliases={0: 0},
    )(main, patches)


def kernel(patches, m):
    del m
    mixed = _tc_tail(_sc_permute(patches), patches)
    ids_b = jnp.arange(_B).reshape(-1, 1)
    target = (ids_b + jnp.arange(_M)) % _B
    mix_target = (ids_b - _M + 1 + jnp.arange(_M * 2 - 1) + _B) % _B
    return (mixed, target, mix_target)
